# baseline (speedup 1.0000x reference)
import functools

import jax
import jax.numpy as jnp
from jax import lax
from jax.experimental import pallas as pl
from jax.experimental.pallas import tpu as pltpu
from jax.experimental.pallas import tpu_sc as plsc

_CTX = 77
_PROMPT = 10
_D = 512
_PC = _D // 128
_N_TAIL = _CTX - 1 - _PROMPT
_TAIL_CHUNKS = 3
_TAIL_TOK = _N_TAIL // _TAIL_CHUNKS


def _build(batch):
    info = plsc.get_sparse_core_info()
    nc, ns = info.num_cores, info.num_subcores
    nw = nc * ns
    assert batch % nw == 0
    bpw = batch // nw
    assert bpw % 3 == 2 and bpw >= 5
    mesh = plsc.VectorSubcoreMesh(core_axis_name="c", subcore_axis_name="s")

    rows = _CTX * _PC

    @functools.partial(
        pl.kernel,
        mesh=mesh,
        out_type=jax.ShapeDtypeStruct((batch * _CTX * _PC, 128), jnp.float32),
        compiler_params=pltpu.CompilerParams(use_tc_tiling_on_sc=False),
        scratch_types=[
            pltpu.VMEM((bpw, _PC), jnp.int32),
            pltpu.VMEM((bpw, _PROMPT * _PC), jnp.int32),
            pltpu.VMEM((bpw, _N_TAIL * _PC), jnp.int32),
            [pltpu.VMEM((rows, 128), jnp.float32)] * 3,
            [pltpu.SemaphoreType.DMA] * 3,
            [pltpu.SemaphoreType.DMA] * 3,
        ],
    )
    def k(idx0_hbm, idx1_hbm, idx2_hbm, pt_hbm, clip_hbm, out_hbm,
          idx0_v, idx1_v, idx2_v, bufs, gsems, osems):
        wid = lax.axis_index("s") * nc + lax.axis_index("c")
        base = wid * bpw
        pltpu.sync_copy(idx0_hbm.at[pl.ds(base, bpw)], idx0_v)
        pltpu.sync_copy(idx1_hbm.at[pl.ds(base, bpw)], idx1_v)
        pltpu.sync_copy(idx2_hbm.at[pl.ds(base, bpw)], idx2_v)

        def fire_gathers(b, p):
            buf, sem = bufs[p], gsems[p]
            pltpu.async_copy(clip_hbm.at[idx0_v.at[b]],
                             buf.at[pl.ds(0, _PC)], sem)
            pltpu.async_copy(pt_hbm.at[idx1_v.at[b]],
                             buf.at[pl.ds(_PC, _PROMPT * _PC)], sem)
            for c in range(_TAIL_CHUNKS):
                n = _TAIL_TOK * _PC
                pltpu.async_copy(
                    clip_hbm.at[idx2_v.at[b, pl.ds(c * n, n)]],
                    buf.at[pl.ds((1 + _PROMPT) * _PC + c * n, n)], sem)

        def wait_gathers(p):
            pltpu.make_async_copy(
                clip_hbm.at[pl.ds(0, rows)], bufs[p], gsems[p]).wait()

        def fire_out(b, p):
            pltpu.async_copy(
                bufs[p], out_hbm.at[pl.ds((base + b) * rows, rows)], osems[p])

        def wait_out(p):
            pltpu.make_async_copy(
                bufs[p], out_hbm.at[pl.ds(base * rows, rows)], osems[p]).wait()

        fire_gathers(0, 0)

        def step(it, carry):
            j0 = 3 * it
            for o in range(3):
                j = j0 + o
                wait_gathers(o)
                fire_out(j, o)

                @pl.when(j >= 2)
                def _():
                    wait_out((o + 1) % 3)

                fire_gathers(j + 1, (o + 1) % 3)
            return carry

        lax.fori_loop(0, (bpw - 2) // 3, step, 0)

        p = (bpw - 2) % 3
        wait_gathers(p)
        fire_out(bpw - 2, p)
        wait_out((p + 1) % 3)
        fire_gathers(bpw - 1, (p + 1) % 3)
        wait_gathers((p + 1) % 3)
        fire_out(bpw - 1, (p + 1) % 3)
        wait_out((p + 2) % 3)
        wait_out(p)
        wait_out((p + 1) % 3)

    return k


def _piece_indices(tok):
    m = (tok * _PC)[..., None] + jnp.arange(_PC, dtype=jnp.int32)
    return m.reshape(tok.shape[0], -1).astype(jnp.int32)


def kernel(tokens, pt_table, clip_table):
    batch = tokens.shape[0]
    tokens = tokens.astype(jnp.int32)
    idx0 = _piece_indices(tokens[:, 0:1])
    idx1 = _piece_indices(tokens[:, 1:1 + _PROMPT])
    idx2 = _piece_indices(tokens[:, 1 + _PROMPT:_CTX])
    ptl = pt_table.reshape(-1, 128)
    clipl = clip_table.reshape(-1, 128)
    y = _build(batch)(idx0, idx1, idx2, ptl, clipl)
    return y.reshape(batch, _CTX, _D)


# speedup vs baseline: 1.0056x; 1.0056x over previous
import functools

import jax
import jax.numpy as jnp
from jax import lax
from jax.experimental import pallas as pl
from jax.experimental.pallas import tpu as pltpu
from jax.experimental.pallas import tpu_sc as plsc

_CTX = 77
_PROMPT = 10
_D = 512


def _build(batch):
    info = plsc.get_sparse_core_info()
    nc, ns = info.num_cores, info.num_subcores
    nw = nc * ns
    assert batch % nw == 0
    bpw = batch // nw
    assert bpw % 3 == 2 and bpw >= 5
    mesh = plsc.VectorSubcoreMesh(core_axis_name="c", subcore_axis_name="s")

    n_tail = _CTX - 1 - _PROMPT

    @functools.partial(
        pl.kernel,
        mesh=mesh,
        out_type=jax.ShapeDtypeStruct((batch, _CTX, _D), jnp.float32),
        compiler_params=pltpu.CompilerParams(use_tc_tiling_on_sc=False),
        scratch_types=[
            pltpu.VMEM((bpw, 1), jnp.int32),
            pltpu.VMEM((bpw, _PROMPT), jnp.int32),
            pltpu.VMEM((bpw, n_tail), jnp.int32),
            [pltpu.VMEM((_CTX, _D), jnp.float32)] * 3,
            [pltpu.SemaphoreType.DMA] * 3,
            [pltpu.SemaphoreType.DMA] * 3,
        ],
    )
    def k(idx0_hbm, idx1_hbm, idx2_hbm, pt_hbm, clip_hbm, out_hbm,
          idx0_v, idx1_v, idx2_v, bufs, gsems, osems):
        wid = lax.axis_index("s") * nc + lax.axis_index("c")
        base = wid * bpw
        pltpu.sync_copy(idx0_hbm.at[pl.ds(base, bpw)], idx0_v)
        pltpu.sync_copy(idx1_hbm.at[pl.ds(base, bpw)], idx1_v)
        pltpu.sync_copy(idx2_hbm.at[pl.ds(base, bpw)], idx2_v)

        def fire_gathers(b, p):
            buf, sem = bufs[p], gsems[p]
            pltpu.async_copy(
                clip_hbm.at[idx0_v.at[b]], buf.at[pl.ds(0, 1)], sem)
            pltpu.async_copy(
                pt_hbm.at[idx1_v.at[b]], buf.at[pl.ds(1, _PROMPT)], sem)
            pltpu.async_copy(
                clip_hbm.at[idx2_v.at[b]], buf.at[pl.ds(1 + _PROMPT, n_tail)],
                sem)

        def wait_gathers(p):
            pltpu.make_async_copy(
                clip_hbm.at[pl.ds(0, _CTX)], bufs[p], gsems[p]).wait()

        def fire_out(b, p):
            pltpu.async_copy(bufs[p], out_hbm.at[base + b], osems[p])

        def wait_out(p):
            pltpu.make_async_copy(bufs[p], out_hbm.at[base], osems[p]).wait()

        fire_gathers(0, 0)

        def step(it, carry):
            j0 = 3 * it
            for o in range(3):
                j = j0 + o

                @pl.when(j >= 2)
                def _():
                    wait_out((o + 1) % 3)

                fire_gathers(j + 1, (o + 1) % 3)
                wait_gathers(o)
                fire_out(j, o)
            return carry

        lax.fori_loop(0, (bpw - 2) // 3, step, 0)

        p = (bpw - 2) % 3
        wait_gathers(p)
        fire_out(bpw - 2, p)
        wait_out((p + 1) % 3)
        fire_gathers(bpw - 1, (p + 1) % 3)
        wait_gathers((p + 1) % 3)
        fire_out(bpw - 1, (p + 1) % 3)
        wait_out((p + 2) % 3)
        wait_out(p)
        wait_out((p + 1) % 3)

    return k


def kernel(tokens, pt_table, clip_table):
    batch = tokens.shape[0]
    idx0 = tokens[:, 0:1].astype(jnp.int32)
    idx1 = tokens[:, 1:1 + _PROMPT].astype(jnp.int32)
    idx2 = tokens[:, 1 + _PROMPT:_CTX].astype(jnp.int32)
    return _build(batch)(idx0, idx1, idx2, pt_table, clip_table)
